# baseline (speedup 1.0000x reference)
import functools

import jax
import jax.numpy as jnp
from jax import lax
from jax.experimental import pallas as pl
from jax.experimental.pallas import tpu as pltpu
from jax.experimental.pallas import tpu_sc as plsc

_DEPTH = 8
_HIDDEN = 768
_EXPERTS = 64
_B = 4
_S = 8192

_TCB = 2048
_S_A = 2048
_S_SC = 2048
_S_B1 = 2048
_S_B2 = _S - _S_A - _S_SC - _S_B1

_NW = 32
_W_PER_B = _NW // _B
_ROWS_PER_W = _S_SC // _W_PER_B
_CHUNK = 64
_NCHUNK = _ROWS_PER_W // _CHUNK
_LANES = 16
_NVEC = _HIDDEN // _LANES
_GROUPS = 4
_VPG = _NVEC // _GROUPS
_COLS_PG = _VPG * _LANES

_SC_ROW0 = _S_A


def _sc_rowsum_body(hs_hbm, tca_hbm, out_hbm, buf, acc, sem0, sem1):
    del tca_hbm
    wid = lax.axis_index("s") * 2 + lax.axis_index("c")
    b = wid // _W_PER_B
    k = wid % _W_PER_B
    base = b * _S + _SC_ROW0 + k * _ROWS_PER_W
    sems = (sem0, sem1)

    zero = jnp.zeros((_LANES,), jnp.float32)
    for j in range(_NVEC):
        acc[pl.ds(j * _LANES, _LANES)] = zero

    def _start(c, bsel):
        pltpu.async_copy(hs_hbm.at[pl.ds(base + c * _CHUNK, _CHUNK)],
                         buf.at[bsel], sems[bsel])

    _start(0, 0)
    _start(1, 1)

    def super_body(c2, carry):
        for bsel in (0, 1):
            c = c2 * 2 + bsel
            pltpu.make_async_copy(hs_hbm.at[pl.ds(0, _CHUNK)],
                                  buf.at[bsel], sems[bsel]).wait()
            for g in range(_GROUPS):
                col0 = g * _COLS_PG

                def row_body(r, accs):
                    return tuple(
                        accs[j] + buf[bsel, r, pl.ds(col0 + j * _LANES, _LANES)]
                        for j in range(_VPG)
                    )

                init = tuple(
                    acc[pl.ds(col0 + j * _LANES, _LANES)]
                    for j in range(_VPG)
                )
                accs = lax.fori_loop(0, _CHUNK, row_body, init)
                for j in range(_VPG):
                    acc[pl.ds(col0 + j * _LANES, _LANES)] = accs[j]

            @pl.when(c2 < _NCHUNK // 2 - 1)
            def _():
                _start(c + 2, bsel)
        return carry

    lax.fori_loop(0, _NCHUNK // 2, super_body, 0)
    pltpu.sync_copy(acc, out_hbm.at[wid])


@functools.lru_cache(maxsize=1)
def _get_sc_rowsum():
    return pl.kernel(
        _sc_rowsum_body,
        out_type=jax.ShapeDtypeStruct((_NW, _HIDDEN), jnp.float32),
        mesh=plsc.VectorSubcoreMesh(core_axis_name="c", subcore_axis_name="s"),
        scratch_types=[
            pltpu.VMEM((2, _CHUNK, _HIDDEN), jnp.float32),
            pltpu.VMEM((_HIDDEN,), jnp.float32),
            pltpu.SemaphoreType.DMA,
            pltpu.SemaphoreType.DMA,
        ],
    )


def _tc_reduce_body(hs_ref, out_ref):
    c = pl.program_id(1)
    s = jnp.sum(hs_ref[0], axis=0, keepdims=True)[None]

    @pl.when(c == 0)
    def _():
        out_ref[...] = s

    @pl.when(c > 0)
    def _():
        out_ref[...] += s


def _tc_reduce_dep_body(hs_ref, dep_ref, out_ref):
    del dep_ref
    _tc_reduce_body(hs_ref, out_ref)


def _tc_reduce(hidden_states, row0, nrows, name, dep=None):
    nblk = nrows // _TCB
    blk0 = row0 // _TCB
    in_specs = [pl.BlockSpec((1, _TCB, _HIDDEN),
                             lambda b, c: (b, blk0 + c, 0))]
    args = [hidden_states]
    body = _tc_reduce_body
    if dep is not None:
        in_specs.append(pl.BlockSpec(memory_space=pltpu.VMEM))
        args.append(dep)
        body = _tc_reduce_dep_body
    return pl.pallas_call(
        body,
        grid=(_B, nblk),
        in_specs=in_specs,
        out_specs=pl.BlockSpec((1, 1, _HIDDEN), lambda b, c: (b, 0, 0)),
        out_shape=jax.ShapeDtypeStruct((_B, 1, _HIDDEN), jnp.float32),
        name=name,
    )(*args)


def _combine_body(part_ref, tca_ref, out_ref):
    psum = part_ref[...]
    out_ref[...] = (jnp.sum(psum.reshape(_B, _W_PER_B, _HIDDEN), axis=1)
                    + tca_ref[:, 0, :])[:, None, :]


def _combine(partials, tca):
    return pl.pallas_call(
        _combine_body,
        out_shape=jax.ShapeDtypeStruct((_B, 1, _HIDDEN), jnp.float32),
        name="sc_combine",
    )(partials, tca)


def _epilogue_body(cd_ref, psa_ref, tcb1_ref, tcb2_ref, demb_ref, w_ref,
                   b_ref, aux_ref, idx_ref):
    cd = cd_ref[0]
    colsum = (psa_ref[:, 0, :] + tcb1_ref[:, 0, :]
              + tcb2_ref[:, 0, :])
    demb = demb_ref[pl.ds(cd, 1), :]
    mean = (colsum + demb) * (1.0 / (_S + 1))
    logits = lax.dot_general(
        mean, w_ref[...], (((1,), (1,)), ((), ())),
        preferred_element_type=jnp.float32,
    ) + b_ref[...]

    m = jnp.max(logits, axis=1, keepdims=True)
    ex = jnp.exp(logits - m)
    probs = ex / jnp.sum(ex, axis=1, keepdims=True)
    usage = jnp.mean(probs, axis=0, keepdims=True)
    uni = 1.0 / _EXPERTS
    kl = jnp.sum(uni * (jnp.log(uni) - jnp.log(usage + 1e-9)))
    coeff = 0.01 * (cd.astype(jnp.float32) / _DEPTH)
    aux_ref[...] = jnp.reshape(coeff * kl, (1, 1))

    iota_e = lax.broadcasted_iota(jnp.int32, (_B, _EXPERTS), 1)
    rmax = jnp.max(logits, axis=1, keepdims=True)
    ei = jnp.min(jnp.where(logits == rmax, iota_e, _EXPERTS),
                 axis=1, keepdims=True)
    counts = jnp.sum((ei == iota_e).astype(jnp.int32),
                     axis=0, keepdims=True)
    cmax = jnp.max(counts)
    idx_ref[...] = jnp.reshape(
        jnp.min(jnp.where(counts == cmax,
                          lax.broadcasted_iota(jnp.int32, (1, _EXPERTS), 1),
                          _EXPERTS)),
        (1, 1))


def _epilogue(cd, psum_a, tcb1, tcb2, demb, w, b):
    return pl.pallas_call(
        _epilogue_body,
        out_shape=(
            jax.ShapeDtypeStruct((1, 1), jnp.float32),
            jax.ShapeDtypeStruct((1, 1), jnp.int32),
        ),
        in_specs=[pl.BlockSpec(memory_space=pltpu.SMEM)]
        + [pl.BlockSpec(memory_space=pltpu.VMEM)] * 6,
        name="router_epilogue",
    )(cd, psum_a, tcb1, tcb2, demb, w, b)


def kernel(hidden_states, current_depth, depth_embedding, router_W, router_b):
    hs_flat = hidden_states.reshape(_B * _S, _HIDDEN)
    tca = _tc_reduce(hidden_states, 0, _S_A, "tc_reduce_a")
    partials = _get_sc_rowsum()(hs_flat, tca)
    tcb1 = _tc_reduce(hidden_states, _S_A + _S_SC, _S_B1, "tc_reduce_b1")
    psum_a = _combine(partials, tca)
    tcb2 = _tc_reduce(hidden_states, _S_A + _S_SC + _S_B1, _S_B2,
                      "tc_reduce_b2", dep=psum_a)
    cd = jnp.asarray(current_depth, jnp.int32).reshape(1)
    aux, nidx = _epilogue(cd, psum_a, tcb1, tcb2, depth_embedding, router_W,
                          router_b.reshape(1, _EXPERTS))
    return aux[0, 0], nidx[0, 0]


# speedup vs baseline: 1.1846x; 1.1846x over previous
import functools

import jax
import jax.numpy as jnp
from jax import lax
from jax.experimental import pallas as pl
from jax.experimental.pallas import tpu as pltpu
from jax.experimental.pallas import tpu_sc as plsc

_DEPTH = 8
_HIDDEN = 768
_EXPERTS = 64
_B = 4
_S = 8192

_NCORES = 1
_NSUB = 16
_NW = _NCORES * _NSUB
_S_SC = 1024
_S_TC = _S - _S_SC
_TCB = 1792

_W_PER_B = _NW // _B
_ROWS_PER_W = _S_SC // _W_PER_B
_CHUNK = 32
_NCHUNK = _ROWS_PER_W // _CHUNK
_LANES = 16
_NVEC = _HIDDEN // _LANES
_GROUPS = 4
_VPG = _NVEC // _GROUPS
_COLS_PG = _VPG * _LANES


def _sc_rowsum_body(hs_hbm, out_hbm, buf, acc, sem0, sem1):
    wid = lax.axis_index("s") * _NCORES + lax.axis_index("c")
    b = wid // _W_PER_B
    k = wid % _W_PER_B
    base = b * _S + _S_TC + k * _ROWS_PER_W
    sems = (sem0, sem1)

    zero = jnp.zeros((_LANES,), jnp.float32)
    for j in range(_NVEC):
        acc[pl.ds(j * _LANES, _LANES)] = zero

    def _start(c, bsel):
        pltpu.async_copy(hs_hbm.at[pl.ds(base + c * _CHUNK, _CHUNK)],
                         buf.at[bsel], sems[bsel])

    _start(0, 0)
    _start(1, 1)

    def super_body(c2, carry):
        for bsel in (0, 1):
            c = c2 * 2 + bsel
            pltpu.make_async_copy(hs_hbm.at[pl.ds(0, _CHUNK)],
                                  buf.at[bsel], sems[bsel]).wait()
            for g in range(_GROUPS):
                col0 = g * _COLS_PG

                def row_body(r, accs):
                    return tuple(
                        accs[j] + buf[bsel, r, pl.ds(col0 + j * _LANES, _LANES)]
                        for j in range(_VPG)
                    )

                init = tuple(
                    acc[pl.ds(col0 + j * _LANES, _LANES)]
                    for j in range(_VPG)
                )
                accs = lax.fori_loop(0, _CHUNK, row_body, init)
                for j in range(_VPG):
                    acc[pl.ds(col0 + j * _LANES, _LANES)] = accs[j]

            @pl.when(c2 < _NCHUNK // 2 - 1)
            def _():
                _start(c + 2, bsel)
        return carry

    lax.fori_loop(0, _NCHUNK // 2, super_body, 0)
    pltpu.sync_copy(acc, out_hbm.at[wid])


@functools.lru_cache(maxsize=1)
def _get_sc_rowsum():
    return pl.kernel(
        _sc_rowsum_body,
        out_type=jax.ShapeDtypeStruct((_NW, _HIDDEN), jnp.float32),
        mesh=plsc.VectorSubcoreMesh(core_axis_name="c", subcore_axis_name="s",
                                    num_cores=_NCORES),
        scratch_types=[
            pltpu.VMEM((2, _CHUNK, _HIDDEN), jnp.float32),
            pltpu.VMEM((_HIDDEN,), jnp.float32),
            pltpu.SemaphoreType.DMA,
            pltpu.SemaphoreType.DMA,
        ],
    )


def _tc_reduce_body(hs_ref, out_ref):
    c = pl.program_id(1)
    s = jnp.sum(hs_ref[0], axis=0, keepdims=True)[None]

    @pl.when(c == 0)
    def _():
        out_ref[...] = s

    @pl.when(c > 0)
    def _():
        out_ref[...] += s


def _tc_reduce(hidden_states):
    return pl.pallas_call(
        _tc_reduce_body,
        grid=(_B, _S_TC // _TCB),
        in_specs=[pl.BlockSpec((1, _TCB, _HIDDEN),
                               lambda b, c: (b, c, 0))],
        out_specs=pl.BlockSpec((1, 1, _HIDDEN), lambda b, c: (b, 0, 0)),
        out_shape=jax.ShapeDtypeStruct((_B, 1, _HIDDEN), jnp.float32),
        name="tc_reduce",
    )(hidden_states)


def _epilogue_body(cd_ref, part_ref, tc_ref, demb_ref, w_ref, b_ref,
                   aux_ref, idx_ref):
    cd = cd_ref[0]
    psum = part_ref[...]
    colsum = (jnp.sum(psum.reshape(_B, _W_PER_B, _HIDDEN), axis=1)
              + tc_ref[:, 0, :])
    demb = demb_ref[pl.ds(cd, 1), :]
    mean = (colsum + demb) * (1.0 / (_S + 1))
    logits = lax.dot_general(
        mean, w_ref[...], (((1,), (1,)), ((), ())),
        preferred_element_type=jnp.float32,
    ) + b_ref[...]

    m = jnp.max(logits, axis=1, keepdims=True)
    ex = jnp.exp(logits - m)
    probs = ex / jnp.sum(ex, axis=1, keepdims=True)
    usage = jnp.mean(probs, axis=0, keepdims=True)
    uni = 1.0 / _EXPERTS
    kl = jnp.sum(uni * (jnp.log(uni) - jnp.log(usage + 1e-9)))
    coeff = 0.01 * (cd.astype(jnp.float32) / _DEPTH)
    aux_ref[...] = jnp.reshape(coeff * kl, (1, 1))

    iota_e = lax.broadcasted_iota(jnp.int32, (_B, _EXPERTS), 1)
    rmax = jnp.max(logits, axis=1, keepdims=True)
    ei = jnp.min(jnp.where(logits == rmax, iota_e, _EXPERTS),
                 axis=1, keepdims=True)
    counts = jnp.sum((ei == iota_e).astype(jnp.int32),
                     axis=0, keepdims=True)
    cmax = jnp.max(counts)
    idx_ref[...] = jnp.reshape(
        jnp.min(jnp.where(counts == cmax,
                          lax.broadcasted_iota(jnp.int32, (1, _EXPERTS), 1),
                          _EXPERTS)),
        (1, 1))


def _epilogue(cd, partials, tcsum, demb, w, b):
    return pl.pallas_call(
        _epilogue_body,
        out_shape=(
            jax.ShapeDtypeStruct((1, 1), jnp.float32),
            jax.ShapeDtypeStruct((1, 1), jnp.int32),
        ),
        in_specs=[pl.BlockSpec(memory_space=pltpu.SMEM)]
        + [pl.BlockSpec(memory_space=pltpu.VMEM)] * 5,
        name="router_epilogue",
    )(cd, partials, tcsum, demb, w, b)


def kernel(hidden_states, current_depth, depth_embedding, router_W, router_b):
    hs_flat = hidden_states.reshape(_B * _S, _HIDDEN)
    partials = _get_sc_rowsum()(hs_flat)
    tcsum = _tc_reduce(hidden_states)
    cd = jnp.asarray(current_depth, jnp.int32).reshape(1)
    aux, nidx = _epilogue(cd, partials, tcsum, depth_embedding, router_W,
                          router_b.reshape(1, _EXPERTS))
    return aux[0, 0], nidx[0, 0]


# speedup vs baseline: 1.1906x; 1.0050x over previous
import functools

import jax
import jax.numpy as jnp
from jax import lax
from jax.experimental import pallas as pl
from jax.experimental.pallas import tpu as pltpu
from jax.experimental.pallas import tpu_sc as plsc

_DEPTH = 8
_HIDDEN = 768
_EXPERTS = 64
_B = 4
_S = 8192

_NCORES = 1
_NSUB = 16
_NW = _NCORES * _NSUB
_S_SC = 1024
_S_TC = _S - _S_SC
_TCB = 1792

_W_PER_B = _NW // _B
_ROWS_PER_W = _S_SC // _W_PER_B
_CHUNK = 64
_NCHUNK = _ROWS_PER_W // _CHUNK
_LANES = 16
_NVEC = _HIDDEN // _LANES
_GROUPS = 4
_VPG = _NVEC // _GROUPS
_COLS_PG = _VPG * _LANES


def _sc_rowsum_body(hs_hbm, out_hbm, buf, acc, sem0, sem1):
    wid = lax.axis_index("s") * _NCORES + lax.axis_index("c")
    b = wid // _W_PER_B
    k = wid % _W_PER_B
    base = b * _S + _S_TC + k * _ROWS_PER_W
    sems = (sem0, sem1)

    zero = jnp.zeros((_LANES,), jnp.float32)
    for j in range(_NVEC):
        acc[pl.ds(j * _LANES, _LANES)] = zero

    def _start(c, bsel):
        pltpu.async_copy(hs_hbm.at[pl.ds(base + c * _CHUNK, _CHUNK)],
                         buf.at[bsel], sems[bsel])

    _start(0, 0)
    _start(1, 1)

    def super_body(c2, carry):
        for bsel in (0, 1):
            c = c2 * 2 + bsel
            pltpu.make_async_copy(hs_hbm.at[pl.ds(0, _CHUNK)],
                                  buf.at[bsel], sems[bsel]).wait()
            for g in range(_GROUPS):
                col0 = g * _COLS_PG

                def row_body(r, accs):
                    return tuple(
                        accs[j] + buf[bsel, r, pl.ds(col0 + j * _LANES, _LANES)]
                        for j in range(_VPG)
                    )

                init = tuple(
                    acc[pl.ds(col0 + j * _LANES, _LANES)]
                    for j in range(_VPG)
                )
                accs = lax.fori_loop(0, _CHUNK, row_body, init)
                for j in range(_VPG):
                    acc[pl.ds(col0 + j * _LANES, _LANES)] = accs[j]

            @pl.when(c2 < _NCHUNK // 2 - 1)
            def _():
                _start(c + 2, bsel)
        return carry

    lax.fori_loop(0, _NCHUNK // 2, super_body, 0)
    pltpu.sync_copy(acc, out_hbm.at[wid])


@functools.lru_cache(maxsize=1)
def _get_sc_rowsum():
    return pl.kernel(
        _sc_rowsum_body,
        out_type=jax.ShapeDtypeStruct((_NW, _HIDDEN), jnp.float32),
        mesh=plsc.VectorSubcoreMesh(core_axis_name="c", subcore_axis_name="s",
                                    num_cores=_NCORES),
        scratch_types=[
            pltpu.VMEM((2, _CHUNK, _HIDDEN), jnp.float32),
            pltpu.VMEM((_HIDDEN,), jnp.float32),
            pltpu.SemaphoreType.DMA,
            pltpu.SemaphoreType.DMA,
        ],
    )


def _tc_reduce_body(hs_ref, out_ref):
    c = pl.program_id(1)
    s = jnp.sum(hs_ref[0], axis=0, keepdims=True)[None]

    @pl.when(c == 0)
    def _():
        out_ref[...] = s

    @pl.when(c > 0)
    def _():
        out_ref[...] += s


def _tc_reduce(hidden_states):
    return pl.pallas_call(
        _tc_reduce_body,
        grid=(_B, _S_TC // _TCB),
        in_specs=[pl.BlockSpec((1, _TCB, _HIDDEN),
                               lambda b, c: (b, c, 0))],
        out_specs=pl.BlockSpec((1, 1, _HIDDEN), lambda b, c: (b, 0, 0)),
        out_shape=jax.ShapeDtypeStruct((_B, 1, _HIDDEN), jnp.float32),
        name="tc_reduce",
    )(hidden_states)


def _epilogue_body(cd_ref, part_ref, tc_ref, demb_ref, w_ref, b_ref,
                   aux_ref, idx_ref):
    cd = cd_ref[0]
    psum = part_ref[...]
    colsum = (jnp.sum(psum.reshape(_B, _W_PER_B, _HIDDEN), axis=1)
              + tc_ref[:, 0, :])
    demb = demb_ref[pl.ds(cd, 1), :]
    mean = (colsum + demb) * (1.0 / (_S + 1))
    logits = lax.dot_general(
        mean, w_ref[...], (((1,), (1,)), ((), ())),
        preferred_element_type=jnp.float32,
    ) + b_ref[...]

    m = jnp.max(logits, axis=1, keepdims=True)
    ex = jnp.exp(logits - m)
    probs = ex / jnp.sum(ex, axis=1, keepdims=True)
    usage = jnp.mean(probs, axis=0, keepdims=True)
    uni = 1.0 / _EXPERTS
    kl = jnp.sum(uni * (jnp.log(uni) - jnp.log(usage + 1e-9)))
    coeff = 0.01 * (cd.astype(jnp.float32) / _DEPTH)
    aux_ref[...] = jnp.reshape(coeff * kl, (1, 1))

    iota_e = lax.broadcasted_iota(jnp.int32, (_B, _EXPERTS), 1)
    rmax = jnp.max(logits, axis=1, keepdims=True)
    ei = jnp.min(jnp.where(logits == rmax, iota_e, _EXPERTS),
                 axis=1, keepdims=True)
    counts = jnp.sum((ei == iota_e).astype(jnp.int32),
                     axis=0, keepdims=True)
    cmax = jnp.max(counts)
    idx_ref[...] = jnp.reshape(
        jnp.min(jnp.where(counts == cmax,
                          lax.broadcasted_iota(jnp.int32, (1, _EXPERTS), 1),
                          _EXPERTS)),
        (1, 1))


def _epilogue(cd, partials, tcsum, demb, w, b):
    return pl.pallas_call(
        _epilogue_body,
        out_shape=(
            jax.ShapeDtypeStruct((1, 1), jnp.float32),
            jax.ShapeDtypeStruct((1, 1), jnp.int32),
        ),
        in_specs=[pl.BlockSpec(memory_space=pltpu.SMEM)]
        + [pl.BlockSpec(memory_space=pltpu.VMEM)] * 5,
        name="router_epilogue",
    )(cd, partials, tcsum, demb, w, b)


def kernel(hidden_states, current_depth, depth_embedding, router_W, router_b):
    hs_flat = hidden_states.reshape(_B * _S, _HIDDEN)
    partials = _get_sc_rowsum()(hs_flat)
    tcsum = _tc_reduce(hidden_states)
    cd = jnp.asarray(current_depth, jnp.int32).reshape(1)
    aux, nidx = _epilogue(cd, partials, tcsum, depth_embedding, router_W,
                          router_b.reshape(1, _EXPERTS))
    return aux[0, 0], nidx[0, 0]
